# baseline (speedup 1.0000x reference)
import jax
import jax.numpy as jnp
from jax import lax
from jax.experimental import pallas as pl
from jax.experimental.pallas import tpu as pltpu
from jax.experimental.pallas import tpu_sc as plsc

F32 = jnp.float32
I32 = jnp.int32

OUT = 7
NROI = 512
D = 128
NCI = 2
NW = 32
ROIS_PER_W = NROI // NW
GROWS = 16 * OUT

_T3 = float((224.0 * (0.5 - 1e-8)) ** 2)
_T4 = float((224.0 * (1.0 - 1e-8)) ** 2)
_T5 = float((224.0 * (2.0 - 1e-8)) ** 2)


def _axis_params(lo, hi, scale, hf):
    lane = lax.iota(I32, 16)
    pf = jnp.minimum(lane >> 1, 6).astype(F32)
    off = 0.25 + 0.5 * (lane & 1).astype(F32)
    a1 = lo * scale - 0.5
    a2 = hi * scale - 0.5
    binw = (a2 - a1) * F32(1.0 / 7.0)
    s = a1 + (pf + off) * binw
    valid = (s > -1.0) & (s < hf)
    sc = jnp.clip(s, 0.0, hf - 1.0)
    i0 = sc.astype(I32)
    l_ = sc - i0.astype(F32)
    vf = jnp.where(valid, F32(0.5), F32(0.0))
    return i0, vf * (1.0 - l_), vf * l_


def _corner_info(ii, w_lo, w_hi, h, add0, mul, shift, info_i, info_f):
    lane = lax.iota(I32, 16)
    c = (lane >> shift) & 3
    cw = lane & 3
    for k in range(OUT):
        a0 = ii[2 * k]
        b0 = ii[2 * k + 1]
        a1 = jnp.minimum(a0 + 1, h - 1)
        b1 = jnp.minimum(b0 + 1, h - 1)
        sel = jnp.where(c == 0, a0, jnp.where(c == 1, a1,
                        jnp.where(c == 2, b0, b1)))
        wsel = jnp.where(cw == 0, w_lo[2 * k], jnp.where(cw == 1,
                         w_hi[2 * k], jnp.where(cw == 2, w_lo[2 * k + 1],
                                                w_hi[2 * k + 1])))
        info_i[k, :] = (add0 + sel) * mul
        info_f[k, :] = wsel


def _sc_body(tb0, tb1, tb2, tb3, boxes_hbm, out_hbm,
             boxes_v, yinfo_i, yinfo_f, xinfo_i, xinfo_f,
             idx_v, patch_v, out_v, sem0, sem1):
    wid = lax.axis_index("s") * 2 + lax.axis_index("c")
    pltpu.sync_copy(boxes_hbm, boxes_v)
    lane = lax.iota(I32, 16)

    def roi_body(i, _):
        roi = wid * ROIS_PER_W + i
        b = roi >> 8
        bv = boxes_v[roi, :]
        x1 = bv[0]
        y1 = bv[1]
        x2 = bv[2]
        y2 = bv[3]
        area = (x2 - x1) * (y2 - y1)
        lv = ((area >= _T3).astype(I32) + (area >= _T4).astype(I32)
              + (area >= _T5).astype(I32))
        h = jnp.where(lv == 0, 256, jnp.where(lv == 1, 128,
                      jnp.where(lv == 2, 64, 32)))
        hf = h.astype(F32)
        scale = jnp.where(lv == 0, F32(0.25), jnp.where(lv == 1, F32(0.125),
                          jnp.where(lv == 2, F32(0.0625), F32(0.03125))))

        yi, wy_lo, wy_hi = _axis_params(y1, y2, scale, hf)
        xi, wx_lo, wx_hi = _axis_params(x1, x2, scale, hf)
        _corner_info(yi, wy_lo, wy_hi, h, b * h, h * NCI, 2, yinfo_i, yinfo_f)
        _corner_info(xi, wx_lo, wx_hi, h, 0, NCI, 0, xinfo_i, xinfo_f)

        def ci_body(ci, _):
            sems = (sem0, sem1)
            xc = lane & 3
            yc = lane >> 2

            def start(p, ci):
                par = p & 1
                yrow = yinfo_i[p, :] + ci
                for q in range(OUT):
                    idx_v[par, pl.ds(q * 16, 16)] = yrow + xinfo_i[q, :]

                for l, tb in enumerate((tb0, tb1, tb2, tb3)):
                    @pl.when(lv == l)
                    def _gather(tb=tb):
                        pltpu.async_copy(
                            tb.at[idx_v.at[par]],
                            patch_v.at[pl.ds(par * GROWS, GROWS)],
                            sems[par])

            start(0, ci)
            for p in range(OUT):
                par = p & 1
                if p < OUT - 1:
                    start(p + 1, ci)
                pltpu.make_async_copy(tb0.at[idx_v.at[par]],
                                      patch_v.at[pl.ds(par * GROWS, GROWS)],
                                      sems[par]).wait()

                wyv = yinfo_f[p, :]
                wy = (wyv[0], wyv[1], wyv[2], wyv[3])

                def q_body(q, _, wy=wy, p=p, par=par):
                    wxv = xinfo_f[q, :]
                    w = [wy[r >> 2] * wxv[r & 3] for r in range(16)]
                    base = par * GROWS + q * 16
                    for v in range(D // 16):
                        sl = pl.ds(v * 16, 16)
                        terms = [w[r] * patch_v[base + r, sl]
                                 for r in range(16)]
                        while len(terms) > 1:
                            terms = [terms[t] + terms[t + 1]
                                     for t in range(0, len(terms), 2)]
                        out_v[p, q, sl] = terms[0]
                    return 0

                lax.fori_loop(0, OUT, q_body, 0)

            pltpu.sync_copy(out_v, out_hbm.at[roi, :, :, pl.ds(ci * D, D)])
            return 0

        lax.fori_loop(0, NCI, ci_body, 0)
        return 0

    lax.fori_loop(0, ROIS_PER_W, roi_body, 0)


@jax.jit
def _run(tb0, tb1, tb2, tb3, boxes_flat):
    mesh = plsc.VectorSubcoreMesh(core_axis_name="c", subcore_axis_name="s")
    f = pl.kernel(
        _sc_body,
        out_type=jax.ShapeDtypeStruct((NROI, OUT, OUT, 256), F32),
        mesh=mesh,
        scratch_types=[
            pltpu.VMEM((NROI, 16), F32),
            pltpu.VMEM((OUT, 16), I32),
            pltpu.VMEM((OUT, 16), F32),
            pltpu.VMEM((OUT, 16), I32),
            pltpu.VMEM((OUT, 16), F32),
            pltpu.VMEM((2, GROWS), I32),
            pltpu.VMEM((2 * GROWS, D), F32),
            pltpu.VMEM((OUT, OUT, D), F32),
            pltpu.SemaphoreType.DMA,
            pltpu.SemaphoreType.DMA,
        ],
    )
    return f(tb0, tb1, tb2, tb3, boxes_flat)


def kernel(feat2, feat3, feat4, feat5, boxes):
    tbls = [f.transpose(0, 2, 3, 1).reshape(-1, D)
            for f in (feat2, feat3, feat4, feat5)]
    boxes16 = jnp.pad(boxes.reshape(NROI, 4), ((0, 0), (0, 12)))
    out = _run(*tbls, boxes16)
    return out.transpose(0, 3, 1, 2)
